# baseline (speedup 1.0000x reference)
import jax
import jax.numpy as jnp
from jax import lax
from jax.experimental import pallas as pl
from jax.experimental.pallas import tpu as pltpu
from jax.experimental.pallas import tpu_sc as plsc

B = 4
F = 128
C = 2 * F
H = 256
W = 256
L = 16
NW = 32
CPW = C // NW
NBUF = 4
HB = 64
NQ = H // HB
VPR = W // L


def _fill_col(stage_v, i, buf):
    def body(r, _):
        for j in range(VPR):
            buf[r, pl.ds(j * L, L)] = stage_v[i, pl.ds(j * L, L)]
        return 0

    lax.fori_loop(0, HB, body, 0)


def _fill_row(stage_v, i, buf, h0):
    def body(g, _):
        vals = stage_v[i, pl.ds(h0 + g * L, L)]
        for lane in range(L):
            v = jnp.full((L,), vals[lane], jnp.float32)
            for j in range(VPR):
                buf[g * L + lane, pl.ds(j * L, L)] = v
        return 0

    lax.fori_loop(0, HB // L, body, 0)


def _sc_body(colT_hbm, rowT_hbm, out_hbm, stage_v, b0, b1, b2, b3, s0, s1, s2, s3):
    cid = lax.axis_index("c")
    sid = lax.axis_index("s")
    wid = sid * 2 + cid
    ch_base = wid * CPW

    bufs = (b0, b1, b2, b3)
    sems = (s0, s1, s2, s3)
    is_col = ch_base < F

    def drain(q, count):
        for _ in range(count):
            pltpu.make_async_copy(colT_hbm.at[pl.ds(0, HB)], bufs[q], sems[q]).wait()

    @pl.when(is_col)
    def _():
        pltpu.sync_copy(colT_hbm.at[pl.ds(ch_base, CPW)], stage_v)

        def body(t, _):
            for q in range(NBUF):
                pl.when(t > 0)(lambda: drain(q, NQ * B))
                i = NBUF * t + q
                ch = ch_base + i
                _fill_col(stage_v, i, bufs[q])
                for b in range(B):
                    for quarter in range(NQ):
                        pltpu.async_copy(
                            bufs[q],
                            out_hbm.at[b, ch, pl.ds(quarter * HB, HB)],
                            sems[q],
                        )
            return 0

        lax.fori_loop(0, CPW // NBUF, body, 0)
        for q in range(NBUF):
            drain(q, NQ * B)

    @pl.when(jnp.logical_not(is_col))
    def _():
        pltpu.sync_copy(rowT_hbm.at[pl.ds(ch_base - F, CPW)], stage_v)

        def body(t, _):
            for q in range(NBUF):
                pl.when(t > 0)(lambda: drain(q, B))
                u = NBUF * t + q
                i = u // NQ
                ch = ch_base + i
                h0 = (u % NQ) * HB
                _fill_row(stage_v, i, bufs[q], h0)
                for b in range(B):
                    pltpu.async_copy(
                        bufs[q], out_hbm.at[b, ch, pl.ds(h0, HB)], sems[q]
                    )
            return 0

        lax.fori_loop(0, CPW * NQ // NBUF, body, 0)
        for q in range(NBUF):
            drain(q, B)


def kernel(bev_mask, row_embed, col_embed):
    colT = col_embed.T
    rowT = row_embed.T

    run = pl.kernel(
        _sc_body,
        mesh=plsc.VectorSubcoreMesh(core_axis_name="c", subcore_axis_name="s"),
        out_type=jax.ShapeDtypeStruct((B, C, H, W), jnp.float32),
        scratch_types=[
            pltpu.VMEM((CPW, W), jnp.float32),
            pltpu.VMEM((HB, W), jnp.float32),
            pltpu.VMEM((HB, W), jnp.float32),
            pltpu.VMEM((HB, W), jnp.float32),
            pltpu.VMEM((HB, W), jnp.float32),
            pltpu.SemaphoreType.DMA,
            pltpu.SemaphoreType.DMA,
            pltpu.SemaphoreType.DMA,
            pltpu.SemaphoreType.DMA,
        ],
    )
    return run(colT, rowT)


# speedup vs baseline: 1.0638x; 1.0638x over previous
import jax
import jax.numpy as jnp
from jax import lax
from jax.experimental import pallas as pl
from jax.experimental.pallas import tpu as pltpu
from jax.experimental.pallas import tpu_sc as plsc

B = 4
F = 128
C = 2 * F
H = 256
W = 256
L = 16
NW = 32
CPW = C // NW
KPW = CPW // 2
HB = 128
VPR = W // L


def _fill_col(stage_v, i, buf):
    vs = [stage_v[i, pl.ds(j * L, L)] for j in range(VPR)]

    def body(r, _):
        for j in range(VPR):
            buf[r, pl.ds(j * L, L)] = vs[j]
        return 0

    lax.fori_loop(0, HB, body, 0)


def _fill_row(stage_v, i, buf, h0):
    def body(g, _):
        vals = stage_v[i, pl.ds(h0 + g * L, L)]
        for lane in range(L):
            v = jnp.full((L,), vals[lane], jnp.float32)
            for j in range(VPR):
                buf[g * L + lane, pl.ds(j * L, L)] = v
        return 0

    lax.fori_loop(0, HB // L, body, 0)


def _sc_body(colT_hbm, rowT_hbm, out_hbm, stage_v, buf0, buf1, sem0, sem1):
    cid = lax.axis_index("c")
    sid = lax.axis_index("s")
    wid = sid * 2 + cid
    base = wid * KPW

    bufs = (buf0, buf1)
    sems = (sem0, sem1)

    def drain(q, count):
        for _ in range(count):
            pltpu.make_async_copy(colT_hbm, bufs[q], sems[q]).wait()

    pltpu.sync_copy(colT_hbm.at[pl.ds(base, KPW)], stage_v.at[pl.ds(0, KPW)])
    pltpu.sync_copy(rowT_hbm.at[pl.ds(base, KPW)], stage_v.at[pl.ds(KPW, KPW)])

    def body_a(t, _):
        for q in range(2):
            pl.when(t > 0)(lambda: drain(q, 2 * B))
            i = 2 * t + q
            ch = base + i
            _fill_col(stage_v, i, bufs[q])
            for b in range(B):
                for half in range(2):
                    pltpu.async_copy(
                        bufs[q],
                        out_hbm.at[b, ch, pl.ds(half * HB, HB)],
                        sems[q],
                    )
        return 0

    lax.fori_loop(0, KPW // 2, body_a, 0)
    for q in range(2):
        drain(q, 2 * B)

    def body_b(t, _):
        for q in range(2):
            pl.when(t > 0)(lambda: drain(q, B))
            u = 2 * t + q
            i = u // 2
            ch = F + base + i
            h0 = (u % 2) * HB
            _fill_row(stage_v, KPW + i, bufs[q], h0)
            for b in range(B):
                pltpu.async_copy(
                    bufs[q], out_hbm.at[b, ch, pl.ds(h0, HB)], sems[q]
                )
        return 0

    lax.fori_loop(0, KPW, body_b, 0)
    for q in range(2):
        drain(q, B)


def kernel(bev_mask, row_embed, col_embed):
    colT = col_embed.T
    rowT = row_embed.T

    run = pl.kernel(
        _sc_body,
        mesh=plsc.VectorSubcoreMesh(core_axis_name="c", subcore_axis_name="s"),
        out_type=jax.ShapeDtypeStruct((B, C, H, W), jnp.float32),
        scratch_types=[
            pltpu.VMEM((CPW, W), jnp.float32),
            pltpu.VMEM((HB, W), jnp.float32),
            pltpu.VMEM((HB, W), jnp.float32),
            pltpu.SemaphoreType.DMA,
            pltpu.SemaphoreType.DMA,
        ],
    )
    return run(colT, rowT)


# speedup vs baseline: 1.0810x; 1.0161x over previous
import jax
import jax.numpy as jnp
from jax import lax
from jax.experimental import pallas as pl
from jax.experimental.pallas import tpu as pltpu
from jax.experimental.pallas import tpu_sc as plsc

B = 4
F = 128
C = 2 * F
H = 256
W = 256
L = 16
NW = 32
CPW = C // NW
KPW = CPW // 2
HB = 128
VPR = W // L


def _fill_col(stage_v, i, buf):
    vs = [stage_v[i, pl.ds(j * L, L)] for j in range(VPR)]

    def body(r, _):
        for j in range(VPR):
            buf[r, pl.ds(j * L, L)] = vs[j]
        return 0

    lax.fori_loop(0, HB, body, 0)


def _fill_row(stage_v, i, buf, h0):
    def body(g, _):
        vals = stage_v[i, pl.ds(h0 + g * L, L)]
        for lane in range(L):
            v = jnp.full((L,), vals[lane], jnp.float32)
            for j in range(VPR):
                buf[g * L + lane, pl.ds(j * L, L)] = v
        return 0

    lax.fori_loop(0, HB // L, body, 0)


def _sc_body(colT_hbm, rowT_hbm, out_hbm, stage_v, buf0, buf1, sem0, sem1):
    cid = lax.axis_index("c")
    sid = lax.axis_index("s")
    wid = sid * 2 + cid
    base = wid * KPW

    bufs = (buf0, buf1)
    sems = (sem0, sem1)

    def drain(q, count):
        for _ in range(count):
            pltpu.make_async_copy(colT_hbm, bufs[q], sems[q]).wait()

    pltpu.sync_copy(colT_hbm.at[pl.ds(base, KPW)], stage_v.at[pl.ds(0, KPW)])
    pltpu.sync_copy(rowT_hbm.at[pl.ds(base, KPW)], stage_v.at[pl.ds(KPW, KPW)])

    def body_a(t, _):
        for q in range(2):
            pl.when(t > 0)(lambda: drain(q, 2 * B))
            i = 2 * t + q
            ch = base + i
            _fill_col(stage_v, i, bufs[q])
            for b in range(B):
                for half in range(2):
                    pltpu.async_copy(
                        bufs[q],
                        out_hbm.at[b, ch, pl.ds(half * HB, HB)],
                        sems[q],
                    )
        return 0

    lax.fori_loop(0, KPW // 2, body_a, 0)

    def body_b(t, _):
        for q in range(2):
            pl.when(t == 0)(lambda: drain(q, 2 * B))
            pl.when(t > 0)(lambda: drain(q, B))
            u = 2 * t + q
            i = u // 2
            ch = F + base + i
            h0 = (u % 2) * HB
            _fill_row(stage_v, KPW + i, bufs[q], h0)
            for b in range(B):
                pltpu.async_copy(
                    bufs[q], out_hbm.at[b, ch, pl.ds(h0, HB)], sems[q]
                )
        return 0

    lax.fori_loop(0, KPW, body_b, 0)
    for q in range(2):
        drain(q, B)


def kernel(bev_mask, row_embed, col_embed):
    colT = col_embed.T
    rowT = row_embed.T

    run = pl.kernel(
        _sc_body,
        mesh=plsc.VectorSubcoreMesh(core_axis_name="c", subcore_axis_name="s"),
        out_type=jax.ShapeDtypeStruct((B, C, H, W), jnp.float32),
        scratch_types=[
            pltpu.VMEM((CPW, W), jnp.float32),
            pltpu.VMEM((HB, W), jnp.float32),
            pltpu.VMEM((HB, W), jnp.float32),
            pltpu.SemaphoreType.DMA,
            pltpu.SemaphoreType.DMA,
        ],
    )
    return run(colT, rowT)
